# Initial kernel scaffold; baseline (speedup 1.0000x reference)
import jax
import jax.numpy as jnp
from jax.experimental import pallas as pl


def kernel(output, target):
    raise NotImplementedError("write your pallas kernel here")


# speedup vs baseline: 5.4436x; 5.4436x over previous
import jax
import jax.numpy as jnp
from jax.experimental import pallas as pl
from jax.experimental.pallas import tpu as pltpu

_LS = 0.1
_N = 32000
_B = 2048
_CONF = 1.0 - _LS
_SV = _LS / (_N - 1)

_RB = 256
_CB = 6400
_NRB = _B // _RB
_NCB = _N // _CB
_BIGI = 2**30


def _loss_kernel(x_ref, t_ref, out_ref, m_ref, s_ref, rs_ref, val_ref, fnd_ref):
    ri = pl.program_id(0)
    ci = pl.program_id(1)
    x = x_ref[...]
    t = t_ref[...]

    bm = jnp.max(x, axis=1, keepdims=True)
    bsum = jnp.sum(x, axis=1, keepdims=True)

    iota = jax.lax.broadcasted_iota(jnp.int32, (_RB, _CB), 1)
    key = jnp.where(t > 0, iota, _BIGI)
    lmin = jnp.min(key, axis=1, keepdims=True)
    lhas = lmin < _BIGI
    lval = jnp.sum(jnp.where(key == lmin, x, 0.0), axis=1, keepdims=True)

    @pl.when(ci == 0)
    def _init():
        m_ref[...] = bm
        s_ref[...] = jnp.sum(jnp.exp(x - bm), axis=1, keepdims=True)
        rs_ref[...] = bsum
        val_ref[...] = jnp.where(lhas, lval, x[:, 0:1])
        fnd_ref[...] = lhas.astype(jnp.int32)

    @pl.when(ci != 0)
    def _update():
        m_old = m_ref[...]
        new_m = jnp.maximum(m_old, bm)
        s_ref[...] = s_ref[...] * jnp.exp(m_old - new_m) + jnp.sum(
            jnp.exp(x - new_m), axis=1, keepdims=True)
        m_ref[...] = new_m
        rs_ref[...] = rs_ref[...] + bsum
        fnd_old = fnd_ref[...] > 0
        val_ref[...] = jnp.where(jnp.logical_and(jnp.logical_not(fnd_old), lhas),
                                 lval, val_ref[...])
        fnd_ref[...] = jnp.logical_or(fnd_old, lhas).astype(jnp.int32)

    @pl.when(ci == _NCB - 1)
    def _finalize():
        lse = m_ref[...] + jnp.log(s_ref[...])
        row_loss = _SV * (rs_ref[...] - _N * lse) + (_CONF - _SV) * (val_ref[...] - lse)
        blk = -jnp.sum(row_loss, axis=0, keepdims=True) / _B
        prev = jnp.where(ri == 0, jnp.zeros((1, 1), jnp.float32), out_ref[...])
        out_ref[...] = prev + blk


def kernel(output, target):
    res = pl.pallas_call(
        _loss_kernel,
        grid=(_NRB, _NCB),
        in_specs=[
            pl.BlockSpec((_RB, _CB), lambda ri, ci: (ri, ci)),
            pl.BlockSpec((_RB, _CB), lambda ri, ci: (ri, ci)),
        ],
        out_specs=pl.BlockSpec((1, 1), lambda ri, ci: (0, 0)),
        out_shape=jax.ShapeDtypeStruct((1, 1), jnp.float32),
        scratch_shapes=[
            pltpu.VMEM((_RB, 1), jnp.float32),
            pltpu.VMEM((_RB, 1), jnp.float32),
            pltpu.VMEM((_RB, 1), jnp.float32),
            pltpu.VMEM((_RB, 1), jnp.float32),
            pltpu.VMEM((_RB, 1), jnp.int32),
        ],
        compiler_params=pltpu.CompilerParams(
            dimension_semantics=("arbitrary", "arbitrary"),
        ),
    )(output, target)
    return res[0, 0]
